# baseline (speedup 1.0000x reference)
import functools

import jax
import jax.numpy as jnp
from jax import lax
from jax.experimental import pallas as pl
from jax.experimental.pallas import tpu as pltpu
from jax.experimental.pallas import tpu_sc as plsc

CYCLE_LEN = 1440
CHANNEL = 64
BATCH = 1024
SEQ_LEN = 336

_NUM_CORES = 2
_NUM_SUBCORES = 16
_NUM_WORKERS = _NUM_CORES * _NUM_SUBCORES
_B_BLOCKS = 8
_T_QUARTERS = 4
_T_PER_TILE = SEQ_LEN // _T_QUARTERS
_CT = CHANNEL // 8
_LANES = 16
_B_GROUPS = 128 // _LANES


@functools.partial(
    pl.kernel,
    mesh=plsc.VectorSubcoreMesh(core_axis_name="c", subcore_axis_name="s"),
    out_type=jax.ShapeDtypeStruct((SEQ_LEN, _CT, _B_BLOCKS, 8, 128),
                                  jnp.float32),
    compiler_params=pltpu.CompilerParams(needs_layout_passes=False),
    scratch_types=[
        pltpu.VMEM((CYCLE_LEN * CHANNEL,), jnp.float32),
        pltpu.VMEM((128,), jnp.int32),
        pltpu.VMEM((2, _CT, 8, 128), jnp.float32),
        pltpu.SemaphoreType.DMA,
        pltpu.SemaphoreType.DMA,
    ],
)
def _cycle_gather_t(index_hbm, data_hbm, out_hbm, table_v, idx_v, buf,
                    sem0, sem1):
    wid = lax.axis_index("s") * _NUM_CORES + lax.axis_index("c")
    b_t = wid % _B_BLOCKS
    t0 = (wid // _B_BLOCKS) * _T_PER_TILE

    pltpu.sync_copy(data_hbm, table_v)
    pltpu.sync_copy(index_hbm.at[pl.ds(b_t * 128, 128)], idx_v)

    def fill(t, p):
        @pl.loop(0, _B_GROUPS)
        def _per_group(g):
            idxg = idx_v[pl.ds(g * _LANES, _LANES)]
            row = idxg + t
            row = jnp.where(row >= CYCLE_LEN, row - CYCLE_LEN, row)
            addr = row * CHANNEL
            for c in range(CHANNEL):
                v = plsc.load_gather(table_v, [addr + c])
                buf[p, c // 8, c % 8, pl.ds(g * _LANES, _LANES)] = v

    def fire(t, p, sem):
        pltpu.async_copy(buf.at[p], out_hbm.at[t, :, b_t], sem)

    def drain(t, p, sem):
        pltpu.make_async_copy(buf.at[p], out_hbm.at[t, :, b_t], sem).wait()

    fill(t0, 0)
    fire(t0, 0, sem0)
    fill(t0 + 1, 1)
    fire(t0 + 1, 1, sem1)

    @pl.loop(t0 + 2, t0 + _T_PER_TILE, step=2)
    def _per_t(t):
        drain(t, 0, sem0)
        fill(t, 0)
        fire(t, 0, sem0)
        drain(t + 1, 1, sem1)
        fill(t + 1, 1)
        fire(t + 1, 1, sem1)

    drain(t0, 0, sem0)
    drain(t0 + 1, 1, sem1)


def kernel(index, length, data):
    del length
    out5 = _cycle_gather_t(index, data.reshape(-1))
    return jnp.transpose(out5, (2, 4, 0, 1, 3)).reshape(BATCH, SEQ_LEN,
                                                        CHANNEL)


# speedup vs baseline: 8.3913x; 8.3913x over previous
import functools

import jax
import jax.numpy as jnp
from jax import lax
from jax.experimental import pallas as pl
from jax.experimental.pallas import tpu as pltpu
from jax.experimental.pallas import tpu_sc as plsc

CYCLE_LEN = 1440
CHANNEL = 64
BATCH = 1024
SEQ_LEN = 336

_NUM_CORES = 2
_NUM_SUBCORES = 16
_NUM_WORKERS = _NUM_CORES * _NUM_SUBCORES
_B_BLOCKS = 8
_T_QUARTERS = 4
_T_PER_TILE = SEQ_LEN // _T_QUARTERS
_CT = CHANNEL // 8
_LANES = 16
_B_GROUPS = 128 // _LANES
_ROW_PAD = CHANNEL + 1


@functools.partial(
    pl.kernel,
    mesh=plsc.VectorSubcoreMesh(core_axis_name="c", subcore_axis_name="s"),
    out_type=jax.ShapeDtypeStruct((SEQ_LEN, _CT, _B_BLOCKS, 8, 128),
                                  jnp.float32),
    compiler_params=pltpu.CompilerParams(needs_layout_passes=False),
    scratch_types=[
        pltpu.VMEM((CYCLE_LEN * _ROW_PAD,), jnp.float32),
        pltpu.VMEM((128,), jnp.int32),
        pltpu.VMEM((2, _CT, 8, 128), jnp.float32),
        pltpu.SemaphoreType.DMA,
        pltpu.SemaphoreType.DMA,
    ],
)
def _cycle_gather_t(index_hbm, data_hbm, out_hbm, table_v, idx_v, buf,
                    sem0, sem1):
    wid = lax.axis_index("s") * _NUM_CORES + lax.axis_index("c")
    b_t = wid % _B_BLOCKS
    t0 = (wid // _B_BLOCKS) * _T_PER_TILE

    pltpu.sync_copy(data_hbm, table_v)
    pltpu.sync_copy(index_hbm.at[pl.ds(b_t * 128, 128)], idx_v)

    def fill(t, p):
        @plsc.parallel_loop(0, _B_GROUPS, unroll=4)
        def _per_group(g):
            idxg = idx_v[pl.ds(g * _LANES, _LANES)]
            row = idxg + t
            row = jnp.where(row >= CYCLE_LEN, row - CYCLE_LEN, row)
            addr = row * _ROW_PAD
            for c0 in range(0, CHANNEL, 16):
                vs = [plsc.load_gather(table_v, [addr + c0 + j])
                      for j in range(16)]
                for j in range(16):
                    c = c0 + j
                    buf[p, c // 8, c % 8, pl.ds(g * _LANES, _LANES)] = vs[j]

    def fire(t, p, sem):
        pltpu.async_copy(buf.at[p], out_hbm.at[t, :, b_t], sem)

    def drain(t, p, sem):
        pltpu.make_async_copy(buf.at[p], out_hbm.at[t, :, b_t], sem).wait()

    fill(t0, 0)
    fire(t0, 0, sem0)
    fill(t0 + 1, 1)
    fire(t0 + 1, 1, sem1)

    @pl.loop(t0 + 2, t0 + _T_PER_TILE, step=2)
    def _per_t(t):
        drain(t, 0, sem0)
        fill(t, 0)
        fire(t, 0, sem0)
        drain(t + 1, 1, sem1)
        fill(t + 1, 1)
        fire(t + 1, 1, sem1)

    drain(t0, 0, sem0)
    drain(t0 + 1, 1, sem1)


def kernel(index, length, data):
    del length
    data_padded = jnp.pad(data, ((0, 0), (0, _ROW_PAD - CHANNEL)))
    out5 = _cycle_gather_t(index, data_padded.reshape(-1))
    return jnp.transpose(out5, (2, 4, 0, 1, 3)).reshape(BATCH, SEQ_LEN,
                                                        CHANNEL)
